# Initial kernel scaffold; baseline (speedup 1.0000x reference)
import jax
import jax.numpy as jnp
from jax.experimental import pallas as pl


def kernel(x, edge_index, e, u, batch, Wk, bk, Wq, bq, Wu, bu):
    raise NotImplementedError("write your pallas kernel here")


# speedup vs baseline: 4.8361x; 4.8361x over previous
import jax
import jax.numpy as jnp
from jax.experimental import pallas as pl
from jax.experimental.pallas import tpu as pltpu

N = 10000
B = 512
FX = 128
FU = 128
H = 128
FU_OUT = 128

BN = 400
G = N // BN


def _body(batch_ref, x_ref, u_ref, Wk_ref, bk_ref, Wq_ref, bq_ref,
          Wu_ref, bu_ref, out_ref, acc_s, qfull_s):
    g = pl.program_id(0)

    @pl.when(g == 0)
    def _init():
        qfull_s[...] = jnp.dot(u_ref[...], Wq_ref[...],
                               preferred_element_type=jnp.float32) + bq_ref[...]
        acc_s[...] = jnp.zeros_like(acc_s)

    x_blk = x_ref[...]
    k = jnp.dot(x_blk, Wk_ref[...],
                preferred_element_type=jnp.float32) + bk_ref[...]
    b = batch_ref[0, 0, :]
    cols = jax.lax.broadcasted_iota(jnp.int32, (BN, B), 1)
    oh = (b[:, None] == cols).astype(jnp.float32)
    q = jnp.dot(oh, qfull_s[...], preferred_element_type=jnp.float32)
    s = jnp.sum(k * q, axis=1, keepdims=True)
    a = jax.nn.sigmoid(s)
    y = a * x_blk
    acc_s[...] += jax.lax.dot_general(
        oh, y, (((0,), (0,)), ((), ())), preferred_element_type=jnp.float32)

    @pl.when(g == G - 1)
    def _final():
        xagg = acc_s[...]
        out_ref[...] = (
            jnp.dot(xagg, Wu_ref[0:FX, :], preferred_element_type=jnp.float32)
            + jnp.dot(u_ref[...], Wu_ref[FX:FX + FU, :],
                      preferred_element_type=jnp.float32)
            + bu_ref[...])


def kernel(x, edge_index, e, u, batch, Wk, bk, Wq, bq, Wu, bu):
    del edge_index, e
    batch3 = batch.astype(jnp.int32).reshape(G, 1, BN)
    in_specs = [
            pl.BlockSpec((1, 1, BN), lambda g: (g, 0, 0)),
            pl.BlockSpec((BN, FX), lambda g: (g, 0)),
            pl.BlockSpec((B, FU), lambda g: (0, 0)),
            pl.BlockSpec((FX, H), lambda g: (0, 0)),
            pl.BlockSpec((1, H), lambda g: (0, 0)),
            pl.BlockSpec((FU, H), lambda g: (0, 0)),
            pl.BlockSpec((1, H), lambda g: (0, 0)),
            pl.BlockSpec((FX + FU, FU_OUT), lambda g: (0, 0)),
            pl.BlockSpec((1, FU_OUT), lambda g: (0, 0)),
        ]
    return pl.pallas_call(
        _body,
        grid=(G,),
        in_specs=in_specs,
        out_specs=pl.BlockSpec((B, FU_OUT), lambda g: (0, 0)),
        out_shape=jax.ShapeDtypeStruct((B, FU_OUT), jnp.float32),
        scratch_shapes=[
            pltpu.VMEM((B, FX), jnp.float32),
            pltpu.VMEM((B, H), jnp.float32),
        ],
        compiler_params=pltpu.CompilerParams(
            dimension_semantics=("arbitrary",)),
    )(batch3, x, u, Wk, bk.reshape(1, H), Wq, bq.reshape(1, H),
      Wu, bu.reshape(1, FU_OUT))
